# baseline (speedup 1.0000x reference)
import functools

import jax
import jax.numpy as jnp
from jax import lax
from jax.experimental import pallas as pl
from jax.experimental.pallas import tpu as pltpu
from jax.experimental.pallas import tpu_sc as plsc

N_NODES = 10000
D = 128
N_EDGES = 320000
NC = 2
NS = 16
NW = NC * NS
CH = 128
KT = 160
K0 = 60
K1 = KT - K0
KMAX = max(K0, K1)
E_PAD = NS * KT * CH
LANES = 16
NPAD = 10240

_mesh = plsc.VectorSubcoreMesh(core_axis_name="c", subcore_axis_name="s")


@functools.partial(
    pl.kernel,
    out_type=jax.ShapeDtypeStruct((NC, NPAD, D), jnp.float32),
    mesh=_mesh,
    scratch_types=[
        pltpu.VMEM((KMAX, CH), jnp.int32),
        pltpu.VMEM((KMAX, CH), jnp.float32),
        pltpu.VMEM((2, CH), jnp.int32),
        pltpu.VMEM((CH, D), jnp.float32),
        pltpu.VMEM_SHARED((NPAD, D), jnp.float32),
    ],
)
def _sc_edge_agg(y_hbm, sd_hbm, w_hbm, out_hbm,
                 sd_v, w_v, idx_v, rows_v, acc):
    c = lax.axis_index("c")
    s = lax.axis_index("s")

    def _zero_row(e, carry):
        for t in range(D // LANES):
            rows_v[e, pl.ds(t * LANES, LANES)] = jnp.zeros((LANES,),
                                                           jnp.float32)
        return carry
    lax.fori_loop(0, CH, _zero_row, 0)

    rpt = NPAD // NS
    base = s * rpt
    for r in range(rpt // CH):
        pltpu.sync_copy(rows_v, acc.at[pl.ds(base + r * CH, CH)])
    plsc.subcore_barrier()

    pltpu.sync_copy(sd_hbm.at[c, s], sd_v)
    pltpu.sync_copy(w_hbm.at[c, s], w_v)

    def _chunk(j, carry):
        def _unpack(g, inner):
            sl = pl.ds(g * LANES, LANES)
            sd = sd_v[j, sl]
            idx_v[0, sl] = sd & 0x3FFF
            idx_v[1, sl] = lax.shift_right_logical(sd, 14)
            return inner
        lax.fori_loop(0, CH // LANES, _unpack, 0)

        pltpu.sync_copy(y_hbm.at[idx_v.at[0]], rows_v)

        def _scale(g, inner):
            w16 = w_v[j, pl.ds(g * LANES, LANES)]
            for i in range(LANES):
                e = g * LANES + i
                for t in range(D // LANES):
                    sl = pl.ds(t * LANES, LANES)
                    rows_v[e, sl] = rows_v[e, sl] * w16[i]
            return inner
        lax.fori_loop(0, CH // LANES, _scale, 0)

        pltpu.sync_copy(rows_v, acc.at[idx_v.at[1]], add=True)
        return carry

    @pl.when(c == 0)
    def _():
        lax.fori_loop(0, K0, _chunk, 0)

    @pl.when(c != 0)
    def _():
        lax.fori_loop(0, K1, _chunk, 0)
    plsc.subcore_barrier()

    pltpu.sync_copy(acc.at[pl.ds(base, rpt)], out_hbm.at[c, pl.ds(base, rpt)])


BM = 1000


def _mm_body(x_ref, w_ref, o_ref):
    o_ref[...] = jnp.dot(x_ref[...], w_ref[...],
                         preferred_element_type=jnp.float32)


def _tc_mm(x, w):
    return pl.pallas_call(
        _mm_body,
        grid=(N_NODES // BM,),
        in_specs=[pl.BlockSpec((BM, D), lambda i: (i, 0)),
                  pl.BlockSpec((D, D), lambda i: (0, 0))],
        out_specs=pl.BlockSpec((BM, D), lambda i: (i, 0)),
        out_shape=jax.ShapeDtypeStruct((N_NODES, D), jnp.float32),
    )(x, w)


def _mid_body(p_ref, x_ref, b_ref, w1r_ref, w2_ref, w2r_ref, y2_ref, r2_ref):
    h = (p_ref[0] + p_ref[1] + b_ref[...]
         + jnp.dot(x_ref[...], w1r_ref[...],
                   preferred_element_type=jnp.float32))
    h = jnp.where(h > 0, h, jnp.exp(jnp.minimum(h, 0.0)) - 1.0)
    y2_ref[...] = jnp.dot(h, w2_ref[...], preferred_element_type=jnp.float32)
    r2_ref[...] = jnp.dot(h, w2r_ref[...], preferred_element_type=jnp.float32)


def _tc_mid(p, x, b1, w1r, w2, w2r):
    return pl.pallas_call(
        _mid_body,
        grid=(N_NODES // BM,),
        in_specs=[pl.BlockSpec((NC, BM, D), lambda i: (0, i, 0)),
                  pl.BlockSpec((BM, D), lambda i: (i, 0)),
                  pl.BlockSpec((1, D), lambda i: (0, 0)),
                  pl.BlockSpec((D, D), lambda i: (0, 0)),
                  pl.BlockSpec((D, D), lambda i: (0, 0)),
                  pl.BlockSpec((D, D), lambda i: (0, 0))],
        out_specs=[pl.BlockSpec((BM, D), lambda i: (i, 0)),
                   pl.BlockSpec((BM, D), lambda i: (i, 0))],
        out_shape=[jax.ShapeDtypeStruct((N_NODES, D), jnp.float32),
                   jax.ShapeDtypeStruct((N_NODES, D), jnp.float32)],
    )(p, x, b1, w1r, w2, w2r)


def _fin_body(q_ref, r2_ref, b_ref, o_ref):
    o_ref[...] = q_ref[0] + q_ref[1] + r2_ref[...] + b_ref[...]


def _tc_fin(q, r2, b2):
    return pl.pallas_call(
        _fin_body,
        grid=(N_NODES // BM,),
        in_specs=[pl.BlockSpec((NC, BM, D), lambda i: (0, i, 0)),
                  pl.BlockSpec((BM, D), lambda i: (i, 0)),
                  pl.BlockSpec((1, D), lambda i: (0, 0))],
        out_specs=pl.BlockSpec((BM, D), lambda i: (i, 0)),
        out_shape=jax.ShapeDtypeStruct((N_NODES, D), jnp.float32),
    )(q, r2, b2)


def kernel(x, edge_index, edge_weights,
           W1_rel, b1_rel, W1_root, W2_rel, b2_rel, W2_root):
    src = edge_index[0].astype(jnp.int32)
    dst = edge_index[1].astype(jnp.int32)
    w = edge_weights.astype(jnp.float32)
    pad = E_PAD - N_EDGES
    sd = src | (dst << 14)
    n0 = NS * K0 * CH

    def _split(a):
        a = jnp.pad(a, (0, pad))
        a0 = a[:n0].reshape(NS, K0, CH)
        a0 = jnp.pad(a0, ((0, 0), (0, KMAX - K0), (0, 0)))
        a1 = a[n0:].reshape(NS, K1, CH)
        a1 = jnp.pad(a1, ((0, 0), (0, KMAX - K1), (0, 0)))
        return jnp.stack([a0, a1])

    sd_m = _split(sd)
    w_m = _split(w)
    b1r = b1_rel.reshape(1, D)
    b2r = b2_rel.reshape(1, D)

    y1 = _tc_mm(x, W1_rel)
    p1 = _sc_edge_agg(y1, sd_m, w_m)
    y2, r2 = _tc_mid(p1, x, b1r, W1_root, W2_rel, W2_root)
    p2 = _sc_edge_agg(y2, sd_m, w_m)
    return _tc_fin(p2, r2, b2r)


# speedup vs baseline: 1.5979x; 1.5979x over previous
import functools

import jax
import jax.numpy as jnp
from jax import lax
from jax.experimental import pallas as pl
from jax.experimental.pallas import tpu as pltpu
from jax.experimental.pallas import tpu_sc as plsc

N_NODES = 10000
D = 128
N_EDGES = 320000
NC = 2
NS = 16
NW = NC * NS
CH = 128
K = -(-N_EDGES // (NW * CH))
E_PAD = NW * K * CH
LANES = 16
NPAD = 10240

_mesh = plsc.VectorSubcoreMesh(core_axis_name="c", subcore_axis_name="s")


@functools.partial(
    pl.kernel,
    out_type=jax.ShapeDtypeStruct((NC, NPAD, D), jnp.float32),
    mesh=_mesh,
    scratch_types=[
        pltpu.VMEM((K, CH), jnp.int32),
        pltpu.VMEM((K, CH), jnp.int32),
        pltpu.VMEM((K, CH), jnp.float32),
        pltpu.VMEM((CH, D), jnp.float32),
        pltpu.VMEM_SHARED((NPAD, D), jnp.float32),
    ],
)
def _sc_edge_agg(y_hbm, src_hbm, dst_hbm, w_hbm, out_hbm,
                 src_v, dst_v, w_v, rows_v, acc):
    c = lax.axis_index("c")
    s = lax.axis_index("s")
    wid = c * NS + s

    def _zero_row(e, carry):
        for t in range(D // LANES):
            rows_v[e, pl.ds(t * LANES, LANES)] = jnp.zeros((LANES,),
                                                           jnp.float32)
        return carry
    lax.fori_loop(0, CH, _zero_row, 0)

    rpt = NPAD // NS
    base = s * rpt
    for r in range(rpt // CH):
        pltpu.sync_copy(rows_v, acc.at[pl.ds(base + r * CH, CH)])
    plsc.subcore_barrier()

    pltpu.sync_copy(src_hbm.at[wid], src_v)
    pltpu.sync_copy(dst_hbm.at[wid], dst_v)
    pltpu.sync_copy(w_hbm.at[wid], w_v)

    def _chunk(j, carry):
        pltpu.sync_copy(y_hbm.at[src_v.at[j]], rows_v)

        def _scale(g, inner):
            w16 = w_v[j, pl.ds(g * LANES, LANES)]
            for i in range(LANES):
                e = g * LANES + i
                for t in range(D // LANES):
                    sl = pl.ds(t * LANES, LANES)
                    rows_v[e, sl] = rows_v[e, sl] * w16[i]
            return inner
        lax.fori_loop(0, CH // LANES, _scale, 0)

        pltpu.sync_copy(rows_v, acc.at[dst_v.at[j]], add=True)
        return carry
    lax.fori_loop(0, K, _chunk, 0)
    plsc.subcore_barrier()

    pltpu.sync_copy(acc.at[pl.ds(base, rpt)], out_hbm.at[c, pl.ds(base, rpt)])


BM = 1000


def _mm_body(x_ref, w_ref, o_ref):
    o_ref[...] = jnp.dot(x_ref[...], w_ref[...],
                         preferred_element_type=jnp.float32)


def _tc_mm(x, w):
    return pl.pallas_call(
        _mm_body,
        grid=(N_NODES // BM,),
        in_specs=[pl.BlockSpec((BM, D), lambda i: (i, 0)),
                  pl.BlockSpec((D, D), lambda i: (0, 0))],
        out_specs=pl.BlockSpec((BM, D), lambda i: (i, 0)),
        out_shape=jax.ShapeDtypeStruct((N_NODES, D), jnp.float32),
    )(x, w)


def _mid_body(p_ref, x_ref, b_ref, w1r_ref, w2_ref, w2r_ref, y2_ref, r2_ref):
    h = (p_ref[0] + p_ref[1] + b_ref[...]
         + jnp.dot(x_ref[...], w1r_ref[...],
                   preferred_element_type=jnp.float32))
    h = jnp.where(h > 0, h, jnp.exp(jnp.minimum(h, 0.0)) - 1.0)
    y2_ref[...] = jnp.dot(h, w2_ref[...], preferred_element_type=jnp.float32)
    r2_ref[...] = jnp.dot(h, w2r_ref[...], preferred_element_type=jnp.float32)


def _tc_mid(p, x, b1, w1r, w2, w2r):
    return pl.pallas_call(
        _mid_body,
        grid=(N_NODES // BM,),
        in_specs=[pl.BlockSpec((NC, BM, D), lambda i: (0, i, 0)),
                  pl.BlockSpec((BM, D), lambda i: (i, 0)),
                  pl.BlockSpec((1, D), lambda i: (0, 0)),
                  pl.BlockSpec((D, D), lambda i: (0, 0)),
                  pl.BlockSpec((D, D), lambda i: (0, 0)),
                  pl.BlockSpec((D, D), lambda i: (0, 0))],
        out_specs=[pl.BlockSpec((BM, D), lambda i: (i, 0)),
                   pl.BlockSpec((BM, D), lambda i: (i, 0))],
        out_shape=[jax.ShapeDtypeStruct((N_NODES, D), jnp.float32),
                   jax.ShapeDtypeStruct((N_NODES, D), jnp.float32)],
    )(p, x, b1, w1r, w2, w2r)


def _fin_body(q_ref, r2_ref, b_ref, o_ref):
    o_ref[...] = q_ref[0] + q_ref[1] + r2_ref[...] + b_ref[...]


def _tc_fin(q, r2, b2):
    return pl.pallas_call(
        _fin_body,
        grid=(N_NODES // BM,),
        in_specs=[pl.BlockSpec((NC, BM, D), lambda i: (0, i, 0)),
                  pl.BlockSpec((BM, D), lambda i: (i, 0)),
                  pl.BlockSpec((1, D), lambda i: (0, 0))],
        out_specs=pl.BlockSpec((BM, D), lambda i: (i, 0)),
        out_shape=jax.ShapeDtypeStruct((N_NODES, D), jnp.float32),
    )(q, r2, b2)


def kernel(x, edge_index, edge_weights,
           W1_rel, b1_rel, W1_root, W2_rel, b2_rel, W2_root):
    src = edge_index[0].astype(jnp.int32)
    dst = edge_index[1].astype(jnp.int32)
    w = edge_weights.astype(jnp.float32)
    pad = E_PAD - N_EDGES
    src_m = jnp.pad(src, (0, pad)).reshape(NW, K, CH)
    dst_m = jnp.pad(dst, (0, pad)).reshape(NW, K, CH)
    w_m = jnp.pad(w, (0, pad)).reshape(NW, K, CH)
    b1r = b1_rel.reshape(1, D)
    b2r = b2_rel.reshape(1, D)

    y1 = _tc_mm(x, W1_rel)
    p1 = _sc_edge_agg(y1, src_m, dst_m, w_m)
    y2, r2 = _tc_mid(p1, x, b1r, W1_root, W2_rel, W2_root)
    p2 = _sc_edge_agg(y2, src_m, dst_m, w_m)
    return _tc_fin(p2, r2, b2r)
